# Initial kernel scaffold; baseline (speedup 1.0000x reference)
import jax
import jax.numpy as jnp
from jax.experimental import pallas as pl


def kernel(msa, seq, idx, emb_W, emb_b, emb_q, emb_left, emb_right, emb_state, pos_emb):
    raise NotImplementedError("write your pallas kernel here")


# speedup vs baseline: 3.3471x; 3.3471x over previous
import functools
import jax
import jax.numpy as jnp
from jax import lax
from jax.experimental import pallas as pl
from jax.experimental.pallas import tpu as pltpu
from jax.experimental.pallas import tpu_sc as plsc

B, N, L = 1, 128, 384
D_INIT, D_MSA, D_PAIR, D_STATE = 48, 256, 128, 32
NBIN = 65
NSEQ = 22

_NW = 32
_ROWS_PER_W = L // _NW


def _sc_body(seq_hbm, idx_hbm, left_hbm, right_hbm, pos_hbm, sttbl_hbm,
             pair_out, state_out,
             seq_v, idx_v, left_v, right_v, pos_v, sttbl_v,
             rowbuf0, rowbuf1, stbuf, sem0, sem1):
    cid = lax.axis_index("c")
    sid = lax.axis_index("s")
    w = sid * 2 + cid
    base = w * _ROWS_PER_W

    pltpu.sync_copy(seq_hbm, seq_v)
    pltpu.sync_copy(idx_hbm, idx_v)
    pltpu.sync_copy(left_hbm, left_v)
    pltpu.sync_copy(right_hbm, right_v)
    pltpu.sync_copy(pos_hbm, pos_v)
    pltpu.sync_copy(sttbl_hbm, sttbl_v)

    iota = lax.iota(jnp.int32, 16)

    @pl.when(w < L // 16)
    def _():
        sbase = w * 16
        for jj in range(16):
            j16 = jnp.full((16,), sbase + jj, jnp.int32)
            sj = plsc.load_gather(seq_v, [j16])
            for cc in range(D_STATE // 16):
                stbuf[jj, pl.ds(cc * 16, 16)] = plsc.load_gather(
                    sttbl_v, [sj, iota + cc * 16])
        pltpu.sync_copy(stbuf, state_out.at[pl.ds(sbase, 16)])

    bufs = (rowbuf0, rowbuf1)
    sems = (sem0, sem1)
    pending = [None, None]
    for rr in range(_ROWS_PER_W):
        i = base + rr
        k = rr % 2
        if pending[k] is not None:
            pending[k].wait()
        buf = bufs[k]
        i16 = jnp.full((16,), i, jnp.int32)
        si = plsc.load_gather(seq_v, [i16])
        di = plsc.load_gather(idx_v, [i16])
        rrow = [plsc.load_gather(right_v, [si, iota + 16 * cc])
                for cc in range(D_PAIR // 16)]

        def jbody(j, carry, buf=buf, di=di, rrow=rrow):
            j16 = jnp.full((16,), j, jnp.int32)
            sj = plsc.load_gather(seq_v, [j16])
            dj = plsc.load_gather(idx_v, [j16])
            pidx = jnp.clip(dj - di + 32, 0, NBIN - 1)
            for cc in range(D_PAIR // 16):
                lv = plsc.load_gather(left_v, [sj, iota + 16 * cc])
                pv = plsc.load_gather(pos_v, [pidx, iota + 16 * cc])
                buf[j, pl.ds(16 * cc, 16)] = lv + pv + rrow[cc]
            return carry

        lax.fori_loop(0, L, jbody, 0)
        pending[k] = pltpu.async_copy(buf, pair_out.at[i], sems[k])
    pending[0].wait()
    pending[1].wait()


def _sc_pair_state(seq, idx, emb_left, emb_right, pos_emb, emb_state):
    mesh = plsc.VectorSubcoreMesh(core_axis_name="c", subcore_axis_name="s")
    kern = pl.kernel(
        _sc_body,
        out_type=[
            jax.ShapeDtypeStruct((L, L, D_PAIR), jnp.float32),
            jax.ShapeDtypeStruct((L, D_STATE), jnp.float32),
        ],
        mesh=mesh,
        compiler_params=pltpu.CompilerParams(needs_layout_passes=False),
        scratch_types=[
            pltpu.VMEM((L,), jnp.int32),
            pltpu.VMEM((L,), jnp.int32),
            pltpu.VMEM((NSEQ, D_PAIR), jnp.float32),
            pltpu.VMEM((NSEQ, D_PAIR), jnp.float32),
            pltpu.VMEM((NBIN, D_PAIR), jnp.float32),
            pltpu.VMEM((NSEQ, D_STATE), jnp.float32),
            pltpu.VMEM((L, D_PAIR), jnp.float32),
            pltpu.VMEM((L, D_PAIR), jnp.float32),
            pltpu.VMEM((16, D_STATE), jnp.float32),
            pltpu.SemaphoreType.DMA,
            pltpu.SemaphoreType.DMA,
        ],
    )
    return kern(seq, idx, emb_left, emb_right, pos_emb, emb_state)


_N_BLK = 8


def _tc_body(seq_ref, msa_ref, w_ref, b_ref, q_ref, out_ref, qrow):
    n = pl.program_id(0)

    @pl.when(n == 0)
    def _():
        seq = seq_ref[...]
        onehot = (seq == lax.broadcasted_iota(jnp.int32, (L, NSEQ), 1)
                  ).astype(jnp.float32)
        qrow[...] = (jnp.dot(onehot, q_ref[...],
                             preferred_element_type=jnp.float32)
                     + b_ref[...])

    x = msa_ref[...]
    y = lax.dot_general(x, w_ref[...], (((2,), (1,)), ((), ())),
                        preferred_element_type=jnp.float32)
    out_ref[...] = y + qrow[...][None]


def _tc_msa(seq2d, msa3, emb_W, emb_b, emb_q):
    grid = (N // _N_BLK,)
    return pl.pallas_call(
        _tc_body,
        grid=grid,
        in_specs=[
            pl.BlockSpec((L, 1), lambda n: (0, 0)),
            pl.BlockSpec((_N_BLK, L, D_INIT), lambda n: (n, 0, 0)),
            pl.BlockSpec((D_MSA, D_INIT), lambda n: (0, 0)),
            pl.BlockSpec((1, D_MSA), lambda n: (0, 0)),
            pl.BlockSpec((NSEQ, D_MSA), lambda n: (0, 0)),
        ],
        out_specs=pl.BlockSpec((_N_BLK, L, D_MSA), lambda n: (n, 0, 0)),
        out_shape=jax.ShapeDtypeStruct((N, L, D_MSA), jnp.float32),
        scratch_shapes=[pltpu.VMEM((L, D_MSA), jnp.float32)],
    )(seq2d, msa3, emb_W, emb_b, emb_q)


@jax.jit
def kernel(msa, seq, idx, emb_W, emb_b, emb_q, emb_left, emb_right,
           emb_state, pos_emb):
    seq1 = seq.reshape(L).astype(jnp.int32)
    idx1 = idx.reshape(L).astype(jnp.int32)

    pair, state = _sc_pair_state(seq1, idx1, emb_left, emb_right, pos_emb,
                                 emb_state)
    msa_e = _tc_msa(seq1.reshape(L, 1), msa.reshape(N, L, D_INIT),
                    emb_W, emb_b.reshape(1, D_MSA), emb_q)

    return (msa_e.reshape(B, N, L, D_MSA),
            pair.reshape(B, L, L, D_PAIR),
            state.reshape(B, L, D_STATE))


# speedup vs baseline: 10.5903x; 3.1640x over previous
import functools
import jax
import jax.numpy as jnp
from jax import lax
from jax.experimental import pallas as pl
from jax.experimental.pallas import tpu as pltpu
from jax.experimental.pallas import tpu_sc as plsc

B, N, L = 1, 128, 384
D_INIT, D_MSA, D_PAIR, D_STATE = 48, 256, 128, 32
NBIN = 65
NSEQ = 22

_NW = 32
_ROWS_PER_W = L // _NW


def _sc_body(seq_hbm, idx_hbm, left_hbm, right_hbm, pos_hbm, sttbl_hbm,
             pair_out, state_out,
             seq_v, idx_v, left_v, right_v, pos_v, sttbl_v,
             rowbuf0, rowbuf1, stbuf, sem0, sem1):
    cid = lax.axis_index("c")
    sid = lax.axis_index("s")
    w = sid * 2 + cid
    base = w * _ROWS_PER_W

    pltpu.sync_copy(seq_hbm, seq_v)
    pltpu.sync_copy(idx_hbm, idx_v)
    pltpu.sync_copy(left_hbm, left_v)
    pltpu.sync_copy(right_hbm, right_v)
    pltpu.sync_copy(pos_hbm, pos_v)
    pltpu.sync_copy(sttbl_hbm, sttbl_v)

    iota = lax.iota(jnp.int32, 16)

    @pl.when(w < L // 16)
    def _():
        sbase = w * 16
        for jj in range(16):
            j16 = jnp.full((16,), sbase + jj, jnp.int32)
            sj = plsc.load_gather(seq_v, [j16])
            for cc in range(D_STATE // 16):
                stbuf[jj, pl.ds(cc * 16, 16)] = plsc.load_gather(
                    sttbl_v, [sj, iota + cc * 16])
        pltpu.sync_copy(stbuf, state_out.at[pl.ds(sbase, 16)])

    bufs = (rowbuf0, rowbuf1)
    sems = (sem0, sem1)
    pending = [None, None]
    for rr in range(_ROWS_PER_W):
        i = base + rr
        k = rr % 2
        if pending[k] is not None:
            pending[k].wait()
        buf = bufs[k]
        i16 = jnp.full((16,), i, jnp.int32)
        si = plsc.load_gather(seq_v, [i16])
        di = plsc.load_gather(idx_v, [i16])
        rrow = [plsc.load_gather(right_v, [si, iota + 16 * cc])
                for cc in range(D_PAIR // 16)]

        @plsc.parallel_loop(0, L, unroll=2)
        def jbody(j, buf=buf, di=di, rrow=rrow):
            j16 = jnp.full((16,), j, jnp.int32)
            sj = plsc.load_gather(seq_v, [j16])
            dj = plsc.load_gather(idx_v, [j16])
            pidx = jnp.clip(dj - di + 32, 0, NBIN - 1)
            for cc in range(D_PAIR // 16):
                lv = plsc.load_gather(left_v, [sj, iota + 16 * cc])
                pv = plsc.load_gather(pos_v, [pidx, iota + 16 * cc])
                buf[j, pl.ds(16 * cc, 16)] = lv + pv + rrow[cc]

        pending[k] = pltpu.async_copy(buf, pair_out.at[i], sems[k])
    pending[0].wait()
    pending[1].wait()


def _sc_pair_state(seq, idx, emb_left, emb_right, pos_emb, emb_state):
    mesh = plsc.VectorSubcoreMesh(core_axis_name="c", subcore_axis_name="s")
    kern = pl.kernel(
        _sc_body,
        out_type=[
            jax.ShapeDtypeStruct((L, L, D_PAIR), jnp.float32),
            jax.ShapeDtypeStruct((L, D_STATE), jnp.float32),
        ],
        mesh=mesh,
        compiler_params=pltpu.CompilerParams(needs_layout_passes=False),
        scratch_types=[
            pltpu.VMEM((L,), jnp.int32),
            pltpu.VMEM((L,), jnp.int32),
            pltpu.VMEM((NSEQ, D_PAIR), jnp.float32),
            pltpu.VMEM((NSEQ, D_PAIR), jnp.float32),
            pltpu.VMEM((NBIN, D_PAIR), jnp.float32),
            pltpu.VMEM((NSEQ, D_STATE), jnp.float32),
            pltpu.VMEM((L, D_PAIR), jnp.float32),
            pltpu.VMEM((L, D_PAIR), jnp.float32),
            pltpu.VMEM((16, D_STATE), jnp.float32),
            pltpu.SemaphoreType.DMA,
            pltpu.SemaphoreType.DMA,
        ],
    )
    return kern(seq, idx, emb_left, emb_right, pos_emb, emb_state)


_N_BLK = 8


def _tc_body(seq_ref, msa_ref, w_ref, b_ref, q_ref, out_ref, qrow):
    n = pl.program_id(0)

    @pl.when(n == 0)
    def _():
        seq = seq_ref[...]
        onehot = (seq == lax.broadcasted_iota(jnp.int32, (L, NSEQ), 1)
                  ).astype(jnp.float32)
        qrow[...] = (jnp.dot(onehot, q_ref[...],
                             preferred_element_type=jnp.float32)
                     + b_ref[...])

    x = msa_ref[...]
    y = lax.dot_general(x, w_ref[...], (((2,), (1,)), ((), ())),
                        preferred_element_type=jnp.float32)
    out_ref[...] = y + qrow[...][None]


def _tc_msa(seq2d, msa3, emb_W, emb_b, emb_q):
    grid = (N // _N_BLK,)
    return pl.pallas_call(
        _tc_body,
        grid=grid,
        in_specs=[
            pl.BlockSpec((L, 1), lambda n: (0, 0)),
            pl.BlockSpec((_N_BLK, L, D_INIT), lambda n: (n, 0, 0)),
            pl.BlockSpec((D_MSA, D_INIT), lambda n: (0, 0)),
            pl.BlockSpec((1, D_MSA), lambda n: (0, 0)),
            pl.BlockSpec((NSEQ, D_MSA), lambda n: (0, 0)),
        ],
        out_specs=pl.BlockSpec((_N_BLK, L, D_MSA), lambda n: (n, 0, 0)),
        out_shape=jax.ShapeDtypeStruct((N, L, D_MSA), jnp.float32),
        scratch_shapes=[pltpu.VMEM((L, D_MSA), jnp.float32)],
    )(seq2d, msa3, emb_W, emb_b, emb_q)


@jax.jit
def kernel(msa, seq, idx, emb_W, emb_b, emb_q, emb_left, emb_right,
           emb_state, pos_emb):
    seq1 = seq.reshape(L).astype(jnp.int32)
    idx1 = idx.reshape(L).astype(jnp.int32)

    pair, state = _sc_pair_state(seq1, idx1, emb_left, emb_right, pos_emb,
                                 emb_state)
    msa_e = _tc_msa(seq1.reshape(L, 1), msa.reshape(N, L, D_INIT),
                    emb_W, emb_b.reshape(1, D_MSA), emb_q)

    return (msa_e.reshape(B, N, L, D_MSA),
            pair.reshape(B, L, L, D_PAIR),
            state.reshape(B, L, D_STATE))
